# Initial kernel scaffold; baseline (speedup 1.0000x reference)
import jax
import jax.numpy as jnp
from jax.experimental import pallas as pl


def kernel(x):
    raise NotImplementedError("write your pallas kernel here")


# speedup vs baseline: 73.5277x; 73.5277x over previous
import jax
import jax.numpy as jnp
from jax.experimental import pallas as pl

_BATCHER8 = [
    (0, 1), (2, 3), (4, 5), (6, 7),
    (0, 2), (1, 3), (4, 6), (5, 7),
    (1, 2), (5, 6),
    (0, 4), (1, 5), (2, 6), (3, 7),
    (2, 4), (3, 5),
    (1, 2), (3, 4), (5, 6),
]

_BITONIC8 = [
    (0, 4), (1, 5), (2, 6), (3, 7),
    (0, 2), (1, 3), (4, 6), (5, 7),
    (0, 1), (2, 3), (4, 5), (6, 7),
]


def _ce(a, i, j):
    hi = jnp.maximum(a[i], a[j])
    lo = jnp.minimum(a[i], a[j])
    a[i] = hi
    a[j] = lo


def _merge_keep_top(a, b):
    m = [jnp.maximum(a[i], b[7 - i]) for i in range(8)]
    for (i, j) in _BITONIC8:
        _ce(m, i, j)
    return m


def _topk_chunk(v, G):
    a = [v[i * G:(i + 1) * G, :] for i in range(8)]
    for (i, j) in _BATCHER8:
        _ce(a, i, j)
    h = G
    while h > 1:
        h //= 2
        hi = [t[:h, :] for t in a]
        lo = [t[h:, :] for t in a]
        a = _merge_keep_top(hi, lo)
    return a


def _body(x_ref, o_ref):
    s = pl.program_id(1)
    S = x_ref.shape[1]
    a = _topk_chunk(x_ref[0], S // 8)

    @pl.when(s == 0)
    def _():
        for k in range(8):
            o_ref[0, k:k + 1, :] = a[k]

    @pl.when(s != 0)
    def _():
        r = [o_ref[0, k:k + 1, :] for k in range(8)]
        m = _merge_keep_top(a, r)
        for k in range(8):
            o_ref[0, k:k + 1, :] = m[k]


def kernel(x):
    B, S, C = x.shape
    CHUNK = 2048
    n = S // CHUNK
    out = pl.pallas_call(
        _body,
        grid=(B, n),
        in_specs=[pl.BlockSpec((1, CHUNK, C), lambda b, s: (b, s, 0))],
        out_specs=pl.BlockSpec((1, 8, C), lambda b, s: (b, 0, 0)),
        out_shape=jax.ShapeDtypeStruct((B, 8, C), x.dtype),
    )(x)
    return out.reshape(B, 8 * C)


# speedup vs baseline: 83.3766x; 1.1339x over previous
import jax
import jax.numpy as jnp
from jax.experimental import pallas as pl
from jax.experimental.pallas import tpu as pltpu

_BATCHER8 = [
    (0, 1), (2, 3), (4, 5), (6, 7),
    (0, 2), (1, 3), (4, 6), (5, 7),
    (1, 2), (5, 6),
    (0, 4), (1, 5), (2, 6), (3, 7),
    (2, 4), (3, 5),
    (1, 2), (3, 4), (5, 6),
]

_BITONIC8 = [
    (0, 4), (1, 5), (2, 6), (3, 7),
    (0, 2), (1, 3), (4, 6), (5, 7),
    (0, 1), (2, 3), (4, 5), (6, 7),
]


def _ce(a, i, j):
    hi = jnp.maximum(a[i], a[j])
    lo = jnp.minimum(a[i], a[j])
    a[i] = hi
    a[j] = lo


def _merge_keep_top(a, b):
    m = [jnp.maximum(a[i], b[7 - i]) for i in range(8)]
    for (i, j) in _BITONIC8:
        _ce(m, i, j)
    return m


def _body(x_ref, o_ref, st_ref):
    s = pl.program_id(1)
    n = pl.num_programs(1)
    S = x_ref.shape[1]
    ng = S // 64

    init = s == 0
    neg = jnp.float32(-jnp.inf)
    st0 = [jnp.where(init, neg, st_ref[0, k]) for k in range(8)]
    st1 = [jnp.where(init, neg, st_ref[1, k]) for k in range(8)]

    for m in range(ng):
        base = m * 64
        g = [x_ref[0, base + 8 * k:base + 8 * (k + 1), :] for k in range(8)]
        for (i, j) in _BATCHER8:
            _ce(g, i, j)
        if m % 2 == 0:
            st0 = _merge_keep_top(st0, g)
        else:
            st1 = _merge_keep_top(st1, g)

    for k in range(8):
        st_ref[0, k] = st0[k]
        st_ref[1, k] = st1[k]

    @pl.when(s == n - 1)
    def _():
        c = _merge_keep_top(st0, st1)
        for shift in (4, 2, 1):
            rolled = [pltpu.roll(c[k], shift, axis=0) for k in range(8)]
            c = _merge_keep_top(c, rolled)
        for k in range(8):
            o_ref[0, k:k + 1, :] = c[k][0:1, :]


def kernel(x):
    B, S, C = x.shape
    CHUNK = 2048
    n = S // CHUNK
    out = pl.pallas_call(
        _body,
        grid=(B, n),
        in_specs=[pl.BlockSpec((1, CHUNK, C), lambda b, s: (b, s, 0))],
        out_specs=pl.BlockSpec((1, 8, C), lambda b, s: (b, 0, 0)),
        out_shape=jax.ShapeDtypeStruct((B, 8, C), x.dtype),
        scratch_shapes=[pltpu.VMEM((2, 8, 8, C), jnp.float32)],
    )(x)
    return out.reshape(B, 8 * C)


# speedup vs baseline: 150.1230x; 1.8005x over previous
import jax
import jax.numpy as jnp
from jax import lax
from jax.experimental import pallas as pl
from jax.experimental.pallas import tpu as pltpu

_BATCHER8 = [
    (0, 1), (2, 3), (4, 5), (6, 7),
    (0, 2), (1, 3), (4, 6), (5, 7),
    (1, 2), (5, 6),
    (0, 4), (1, 5), (2, 6), (3, 7),
    (2, 4), (3, 5),
    (1, 2), (3, 4), (5, 6),
]

_BITONIC8 = [
    (0, 4), (1, 5), (2, 6), (3, 7),
    (0, 2), (1, 3), (4, 6), (5, 7),
    (0, 1), (2, 3), (4, 5), (6, 7),
]


def _ce(a, i, j):
    hi = jnp.maximum(a[i], a[j])
    lo = jnp.minimum(a[i], a[j])
    a[i] = hi
    a[j] = lo


def _merge_keep_top(a, b):
    m = [jnp.maximum(a[i], b[7 - i]) for i in range(8)]
    for (i, j) in _BITONIC8:
        _ce(m, i, j)
    return m


def _sorted_group(x_ref, b, start):
    g = [x_ref[b, pl.ds(pl.multiple_of(start + 8 * k, 8), 8), :]
         for k in range(8)]
    for (i, j) in _BATCHER8:
        _ce(g, i, j)
    return g


def _body(x_ref, o_ref):
    NB, S, C = x_ref.shape
    niter = S // 128

    for b in range(NB):
        neg = jnp.full((8, C), -jnp.inf, jnp.float32)

        def step(m, carry):
            st0, st1 = carry
            base = m * 128
            st0 = _merge_keep_top(list(st0), _sorted_group(x_ref, b, base))
            st1 = _merge_keep_top(list(st1), _sorted_group(x_ref, b, base + 64))
            return (tuple(st0), tuple(st1))

        st0, st1 = lax.fori_loop(
            0, niter, step, (tuple([neg] * 8), tuple([neg] * 8)))
        c = _merge_keep_top(list(st0), list(st1))
        for shift in (4, 2, 1):
            rolled = [pltpu.roll(c[k], shift, axis=0) for k in range(8)]
            c = _merge_keep_top(c, rolled)
        for k in range(8):
            o_ref[b, k:k + 1, :] = c[k][0:1, :]


def kernel(x):
    B, S, C = x.shape
    NB = 4
    out = pl.pallas_call(
        _body,
        grid=(B // NB,),
        in_specs=[pl.BlockSpec((NB, S, C), lambda b: (b, 0, 0))],
        out_specs=pl.BlockSpec((NB, 8, C), lambda b: (b, 0, 0)),
        out_shape=jax.ShapeDtypeStruct((B, 8, C), x.dtype),
    )(x)
    return out.reshape(B, 8 * C)


# speedup vs baseline: 166.2437x; 1.1074x over previous
import jax
import jax.numpy as jnp
from jax import lax
from jax.experimental import pallas as pl
from jax.experimental.pallas import tpu as pltpu

_BATCHER8 = [
    (0, 1), (2, 3), (4, 5), (6, 7),
    (0, 2), (1, 3), (4, 6), (5, 7),
    (1, 2), (5, 6),
    (0, 4), (1, 5), (2, 6), (3, 7),
    (2, 4), (3, 5),
    (1, 2), (3, 4), (5, 6),
]

_BITONIC8 = [
    (0, 4), (1, 5), (2, 6), (3, 7),
    (0, 2), (1, 3), (4, 6), (5, 7),
    (0, 1), (2, 3), (4, 5), (6, 7),
]


def _ce(a, i, j):
    hi = jnp.maximum(a[i], a[j])
    lo = jnp.minimum(a[i], a[j])
    a[i] = hi
    a[j] = lo


def _merge_keep_top(a, b):
    m = [jnp.maximum(a[i], b[7 - i]) for i in range(8)]
    for (i, j) in _BITONIC8:
        _ce(m, i, j)
    return m


def _sorted_group(x_ref, b, start):
    g = [x_ref[b, pl.ds(pl.multiple_of(start + 8 * k, 8), 8), :]
         for k in range(8)]
    for (i, j) in _BATCHER8:
        _ce(g, i, j)
    return g


def _body(x_ref, o_ref):
    NB, S, C = x_ref.shape
    niter = S // 128

    for b in range(NB):
        neg = jnp.full((8, C), -jnp.inf, jnp.float32)

        def step(m, carry):
            st0, st1 = carry
            base = m * 128
            st0 = _merge_keep_top(list(st0), _sorted_group(x_ref, b, base))
            st1 = _merge_keep_top(list(st1), _sorted_group(x_ref, b, base + 64))
            return (tuple(st0), tuple(st1))

        st0, st1 = lax.fori_loop(
            0, niter, step, (tuple([neg] * 8), tuple([neg] * 8)), unroll=2)
        c = _merge_keep_top(list(st0), list(st1))
        for shift in (4, 2, 1):
            rolled = [pltpu.roll(c[k], shift, axis=0) for k in range(8)]
            c = _merge_keep_top(c, rolled)
        for k in range(8):
            o_ref[b, k:k + 1, :] = c[k][0:1, :]


def kernel(x):
    B, S, C = x.shape
    NB = 4
    out = pl.pallas_call(
        _body,
        grid=(B // NB,),
        in_specs=[pl.BlockSpec((NB, S, C), lambda b: (b, 0, 0))],
        out_specs=pl.BlockSpec((NB, 8, C), lambda b: (b, 0, 0)),
        out_shape=jax.ShapeDtypeStruct((B, 8, C), x.dtype),
    )(x)
    return out.reshape(B, 8 * C)


# speedup vs baseline: 172.6145x; 1.0383x over previous
import jax
import jax.numpy as jnp
from jax import lax
from jax.experimental import pallas as pl
from jax.experimental.pallas import tpu as pltpu

_BATCHER8 = [
    (0, 1), (2, 3), (4, 5), (6, 7),
    (0, 2), (1, 3), (4, 6), (5, 7),
    (1, 2), (5, 6),
    (0, 4), (1, 5), (2, 6), (3, 7),
    (2, 4), (3, 5),
    (1, 2), (3, 4), (5, 6),
]

_BITONIC8 = [
    (0, 4), (1, 5), (2, 6), (3, 7),
    (0, 2), (1, 3), (4, 6), (5, 7),
    (0, 1), (2, 3), (4, 5), (6, 7),
]


def _ce(a, i, j):
    hi = jnp.maximum(a[i], a[j])
    lo = jnp.minimum(a[i], a[j])
    a[i] = hi
    a[j] = lo


def _merge_keep_top(a, b):
    m = [jnp.maximum(a[i], b[7 - i]) for i in range(8)]
    for (i, j) in _BITONIC8:
        _ce(m, i, j)
    return m


def _sorted_group(x_ref, b, start):
    g = [x_ref[b, pl.ds(pl.multiple_of(start + 8 * k, 8), 8), :]
         for k in range(8)]
    for (i, j) in _BATCHER8:
        _ce(g, i, j)
    return g


def _body(x_ref, o_ref):
    NB, S, C = x_ref.shape
    niter = S // 128

    for b in range(NB):
        neg = jnp.full((8, C), -jnp.inf, jnp.float32)

        def step(m, carry):
            st0, st1 = carry
            base = m * 128
            st0 = _merge_keep_top(list(st0), _sorted_group(x_ref, b, base))
            st1 = _merge_keep_top(list(st1), _sorted_group(x_ref, b, base + 64))
            return (tuple(st0), tuple(st1))

        st0, st1 = lax.fori_loop(
            0, niter, step, (tuple([neg] * 8), tuple([neg] * 8)), unroll=4)
        c = _merge_keep_top(list(st0), list(st1))
        for shift in (4, 2, 1):
            rolled = [pltpu.roll(c[k], shift, axis=0) for k in range(8)]
            c = _merge_keep_top(c, rolled)
        for k in range(8):
            o_ref[b, k:k + 1, :] = c[k][0:1, :]


def kernel(x):
    B, S, C = x.shape
    NB = 4
    out = pl.pallas_call(
        _body,
        grid=(B // NB,),
        in_specs=[pl.BlockSpec((NB, S, C), lambda b: (b, 0, 0))],
        out_specs=pl.BlockSpec((NB, 8, C), lambda b: (b, 0, 0)),
        out_shape=jax.ShapeDtypeStruct((B, 8, C), x.dtype),
    )(x)
    return out.reshape(B, 8 * C)
